# Initial kernel scaffold; baseline (speedup 1.0000x reference)
import functools

import jax
import jax.numpy as jnp
from jax import lax
from jax.experimental import pallas as pl
from jax.experimental.pallas import tpu as pltpu
from jax.experimental.pallas import tpu_sc as plsc

_N = 10000
_E = 320000
_D = 128
_G = 512
_EPS = 1e-5

_NC = 2
_NS = 16
_NW = _NC * _NS
_CHUNK = 80
_CHUNKS_PER_TILE = _E // (_NW * _CHUNK)
_ZROWS = 125
_STRIPE = _N // _NS


def _segment_sum_sc(h, src, dst):
    mesh = plsc.VectorSubcoreMesh(core_axis_name="c", subcore_axis_name="s")

    @functools.partial(
        pl.kernel,
        mesh=mesh,
        out_type=jax.ShapeDtypeStruct((_NC, _N, _D), jnp.float32),
        scratch_types=[
            pltpu.VMEM((_CHUNKS_PER_TILE, _CHUNK), jnp.int32),
            pltpu.VMEM((_CHUNKS_PER_TILE, _CHUNK), jnp.int32),
            pltpu.VMEM((_CHUNK, _D), jnp.float32),
            pltpu.VMEM((_ZROWS, _D), jnp.float32),
            pltpu.VMEM_SHARED((_N, _D), jnp.float32),
        ],
    )
    def ksc(h_hbm, src_hbm, dst_hbm, out_hbm, src_v, dst_v, rows_v, zero_v, acc):
        c = lax.axis_index("c")
        s = lax.axis_index("s")
        wid = s * _NC + c

        @pl.loop(0, _ZROWS)
        def _(i):
            @pl.loop(0, _D // 16)
            def _(j):
                zero_v[i, pl.ds(j * 16, 16)] = jnp.zeros((16,), jnp.float32)

        @pl.loop(0, _STRIPE // _ZROWS)
        def _(r):
            pltpu.sync_copy(zero_v, acc.at[pl.ds(s * _STRIPE + r * _ZROWS, _ZROWS)])

        pltpu.sync_copy(src_hbm.at[pl.ds(wid * _CHUNKS_PER_TILE, _CHUNKS_PER_TILE)], src_v)
        pltpu.sync_copy(dst_hbm.at[pl.ds(wid * _CHUNKS_PER_TILE, _CHUNKS_PER_TILE)], dst_v)

        plsc.subcore_barrier()

        @pl.loop(0, _CHUNKS_PER_TILE)
        def _(j):
            pltpu.sync_copy(h_hbm.at[src_v.at[j]], rows_v)
            pltpu.sync_copy(rows_v, acc.at[dst_v.at[j]], add=True)

        plsc.subcore_barrier()

        pltpu.sync_copy(acc.at[pl.ds(s * _STRIPE, _STRIPE)],
                        out_hbm.at[c].at[pl.ds(s * _STRIPE, _STRIPE)])

    return ksc(h, src, dst)


def _gin_layer_tc(h, agg, w1, b1, w2, b2, gamma, beta, relu_out):
    def body(h_ref, a_ref, w1_ref, b1_ref, w2_ref, b2_ref, g_ref, be_ref, o_ref):
        t = h_ref[...] + a_ref[0] + a_ref[1]
        u = jnp.dot(t, w1_ref[...], preferred_element_type=jnp.float32) + b1_ref[...]
        u = jnp.maximum(u, 0.0)
        v = jnp.dot(u, w2_ref[...], preferred_element_type=jnp.float32) + b2_ref[...]
        mean = jnp.mean(v, axis=0, keepdims=True)
        cen = v - mean
        var = jnp.mean(cen * cen, axis=0, keepdims=True)
        o = cen * (g_ref[...] * lax.rsqrt(var + _EPS)) + be_ref[...]
        if relu_out:
            o = jnp.maximum(o, 0.0)
        o_ref[...] = o

    return pl.pallas_call(
        body, out_shape=jax.ShapeDtypeStruct((_N, _D), jnp.float32)
    )(h, agg, w1, b1, w2, b2, gamma, beta)


def _pool_linear_tc(h, batch2d, lin_w, lin_b):
    def body(h_ref, b_ref, w_ref, bias_ref, o_ref):
        ids = lax.broadcasted_iota(jnp.int32, (_N, _G), 1)
        oh = (b_ref[...] == ids).astype(jnp.float32)
        sums = lax.dot_general(oh, h_ref[...], (((0,), (0,)), ((), ())),
                               preferred_element_type=jnp.float32)
        cnt = jnp.sum(oh, axis=0)[:, None]
        pooled = sums / jnp.maximum(cnt, 1.0)
        o_ref[...] = jnp.dot(pooled, w_ref[...],
                             preferred_element_type=jnp.float32) + bias_ref[...]

    return pl.pallas_call(
        body, out_shape=jax.ShapeDtypeStruct((_G, _D), jnp.float32)
    )(h, batch2d, lin_w, lin_b)


def kernel(x, edge_index, batch,
           w1_0, b1_0, w2_0, b2_0, gamma_0, beta_0,
           w1_1, b1_1, w2_1, b2_1, gamma_1, beta_1,
           w1_2, b1_2, w2_2, b2_2, gamma_2, beta_2,
           lin_w, lin_b):
    src = edge_index[0].reshape(_E // _CHUNK, _CHUNK)
    dst = edge_index[1].reshape(_E // _CHUNK, _CHUNK)
    batch2d = batch.reshape(_N, 1)

    layers = [
        (w1_0, b1_0, w2_0, b2_0, gamma_0, beta_0),
        (w1_1, b1_1, w2_1, b2_1, gamma_1, beta_1),
        (w1_2, b1_2, w2_2, b2_2, gamma_2, beta_2),
    ]
    h = x
    for i, (w1, b1, w2, b2, g, be) in enumerate(layers):
        agg = _segment_sum_sc(h, src, dst)
        h = _gin_layer_tc(h, agg, w1, b1.reshape(1, _D), w2, b2.reshape(1, _D),
                          g.reshape(1, _D), be.reshape(1, _D), relu_out=(i < 2))
    return _pool_linear_tc(h, batch2d, lin_w, lin_b)


# speedup vs baseline: 6.5496x; 6.5496x over previous
import functools

import jax
import jax.numpy as jnp
from jax import lax
from jax.experimental import pallas as pl
from jax.experimental.pallas import tpu as pltpu
from jax.experimental.pallas import tpu_sc as plsc

_N = 10000
_E = 320000
_D = 128
_G = 512
_EPS = 1e-5

_NC = 2
_NS = 16
_NW = _NC * _NS
_CHUNK = 80
_CHUNKS_PER_TILE = _E // (_NW * _CHUNK)
_NPAD = 10240
_STRIPE = _NPAD // _NS
_ZROWS = _CHUNK


def _segment_sum_sc(h, src, dst):
    mesh = plsc.VectorSubcoreMesh(core_axis_name="c", subcore_axis_name="s")

    @functools.partial(
        pl.kernel,
        mesh=mesh,
        out_type=jax.ShapeDtypeStruct((_NC, _NPAD, _D), jnp.float32),
        scratch_types=[
            pltpu.VMEM((_CHUNKS_PER_TILE, _CHUNK), jnp.int32),
            pltpu.VMEM((_CHUNKS_PER_TILE, _CHUNK), jnp.int32),
            pltpu.VMEM((_CHUNK, _D), jnp.float32),
            pltpu.VMEM_SHARED((_NPAD, _D), jnp.float32),
        ],
    )
    def ksc(h_hbm, src_hbm, dst_hbm, out_hbm, src_v, dst_v, rows_v, acc):
        c = lax.axis_index("c")
        s = lax.axis_index("s")
        wid = s * _NC + c

        @pl.loop(0, _ZROWS)
        def _(i):
            @pl.loop(0, _D // 16)
            def _(j):
                rows_v[i, pl.ds(j * 16, 16)] = jnp.zeros((16,), jnp.float32)

        base = s * _STRIPE

        @pl.loop(0, _STRIPE // _ZROWS)
        def _(r):
            pltpu.sync_copy(rows_v, acc.at[pl.ds(base + r * _ZROWS, _ZROWS)])

        pltpu.sync_copy(src_hbm.at[wid], src_v)
        pltpu.sync_copy(dst_hbm.at[wid], dst_v)

        plsc.subcore_barrier()

        @pl.loop(0, _CHUNKS_PER_TILE)
        def _(j):
            pltpu.sync_copy(h_hbm.at[src_v.at[j]], rows_v)
            pltpu.sync_copy(rows_v, acc.at[dst_v.at[j]], add=True)

        plsc.subcore_barrier()

        pltpu.sync_copy(acc.at[pl.ds(base, _STRIPE)],
                        out_hbm.at[c].at[pl.ds(base, _STRIPE)])

    return ksc(h, src, dst)


def _gin_layer_tc(h, agg, w1, b1, w2, b2, gamma, beta, relu_out):
    def body(h_ref, a_ref, w1_ref, b1_ref, w2_ref, b2_ref, g_ref, be_ref, o_ref):
        t = h_ref[...] + a_ref[0, :_N] + a_ref[1, :_N]
        u = jnp.dot(t, w1_ref[...], preferred_element_type=jnp.float32) + b1_ref[...]
        u = jnp.maximum(u, 0.0)
        v = jnp.dot(u, w2_ref[...], preferred_element_type=jnp.float32) + b2_ref[...]
        mean = jnp.mean(v, axis=0, keepdims=True)
        cen = v - mean
        var = jnp.mean(cen * cen, axis=0, keepdims=True)
        o = cen * (g_ref[...] * lax.rsqrt(var + _EPS)) + be_ref[...]
        if relu_out:
            o = jnp.maximum(o, 0.0)
        o_ref[...] = o

    return pl.pallas_call(
        body, out_shape=jax.ShapeDtypeStruct((_N, _D), jnp.float32)
    )(h, agg, w1, b1, w2, b2, gamma, beta)


def _pool_linear_tc(h, batch2d, lin_w, lin_b):
    def body(h_ref, b_ref, w_ref, bias_ref, o_ref):
        ids = lax.broadcasted_iota(jnp.int32, (_N, _G), 1)
        oh = (b_ref[...] == ids).astype(jnp.float32)
        sums = lax.dot_general(oh, h_ref[...], (((0,), (0,)), ((), ())),
                               preferred_element_type=jnp.float32,
                               precision=lax.Precision.HIGHEST)
        cnt = jnp.sum(oh, axis=0)[:, None]
        pooled = sums / jnp.maximum(cnt, 1.0)
        o_ref[...] = jnp.dot(pooled, w_ref[...],
                             preferred_element_type=jnp.float32) + bias_ref[...]

    return pl.pallas_call(
        body, out_shape=jax.ShapeDtypeStruct((_G, _D), jnp.float32)
    )(h, batch2d, lin_w, lin_b)


def kernel(x, edge_index, batch,
           w1_0, b1_0, w2_0, b2_0, gamma_0, beta_0,
           w1_1, b1_1, w2_1, b2_1, gamma_1, beta_1,
           w1_2, b1_2, w2_2, b2_2, gamma_2, beta_2,
           lin_w, lin_b):
    src = edge_index[0].reshape(_NW, _CHUNKS_PER_TILE, _CHUNK)
    dst = edge_index[1].reshape(_NW, _CHUNKS_PER_TILE, _CHUNK)
    batch2d = batch.reshape(_N, 1)

    layers = [
        (w1_0, b1_0, w2_0, b2_0, gamma_0, beta_0),
        (w1_1, b1_1, w2_1, b2_1, gamma_1, beta_1),
        (w1_2, b1_2, w2_2, b2_2, gamma_2, beta_2),
    ]
    h = x
    for i, (w1, b1, w2, b2, g, be) in enumerate(layers):
        agg = _segment_sum_sc(h, src, dst)
        h = _gin_layer_tc(h, agg, w1, b1.reshape(1, _D), w2, b2.reshape(1, _D),
                          g.reshape(1, _D), be.reshape(1, _D), relu_out=(i < 2))
    return _pool_linear_tc(h, batch2d, lin_w, lin_b)


# speedup vs baseline: 10.6145x; 1.6206x over previous
import functools

import jax
import jax.numpy as jnp
from jax import lax
from jax.experimental import pallas as pl
from jax.experimental.pallas import tpu as pltpu
from jax.experimental.pallas import tpu_sc as plsc

_N = 10000
_E = 320000
_D = 128
_G = 512
_EPS = 1e-5

_NC = 2
_NS = 16
_NW = _NC * _NS
_CHUNK = 80
_CHUNKS_PER_TILE = _E // (_NW * _CHUNK)
_NPAD = 10240
_STRIPE = _NPAD // _NS
_ZROWS = _CHUNK


def _segment_sum_sc(h, epk):
    mesh = plsc.VectorSubcoreMesh(core_axis_name="c", subcore_axis_name="s")

    @functools.partial(
        pl.kernel,
        mesh=mesh,
        out_type=jax.ShapeDtypeStruct((_NC, _NPAD, _D), jnp.float32),
        scratch_types=[
            pltpu.VMEM((_CHUNKS_PER_TILE, _CHUNK), jnp.int32),
            pltpu.VMEM((2, _CHUNK), jnp.int32),
            pltpu.VMEM((2, _CHUNK), jnp.int32),
            pltpu.VMEM((2, _CHUNK, _D), jnp.float32),
            pltpu.VMEM_SHARED((_NPAD, _D), jnp.float32),
            pltpu.SemaphoreType.DMA,
            pltpu.SemaphoreType.DMA,
        ],
    )
    def ksc(h_hbm, epk_hbm, out_hbm, eidx_v, srcu, dstu, rows_v, acc,
            gsem0, gsem1):
        c = lax.axis_index("c")
        s = lax.axis_index("s")
        wid = s * _NC + c

        @pl.loop(0, _ZROWS)
        def _(i):
            @pl.loop(0, _D // 16)
            def _(j):
                rows_v[0, i, pl.ds(j * 16, 16)] = jnp.zeros((16,), jnp.float32)

        base = s * _STRIPE

        @pl.loop(0, _STRIPE // _ZROWS)
        def _(r):
            pltpu.sync_copy(rows_v.at[0], acc.at[pl.ds(base + r * _ZROWS, _ZROWS)])

        pltpu.sync_copy(epk_hbm.at[wid], eidx_v)

        plsc.subcore_barrier()

        sems = (gsem0, gsem1)

        def unpack(j, b):
            @pl.loop(0, _CHUNK // 16)
            def _(k):
                p = eidx_v[j, pl.ds(k * 16, 16)]
                srcu[b, pl.ds(k * 16, 16)] = p & 0x3FFF
                dstu[b, pl.ds(k * 16, 16)] = p >> 14

        def g_copy(b):
            return pltpu.make_async_copy(
                h_hbm.at[srcu.at[b]], rows_v.at[b], sems[b])

        def s_sync(b):
            pltpu.sync_copy(rows_v.at[b], acc.at[dstu.at[b]], add=True)

        unpack(0, 0)
        g_copy(0).start()

        @pl.loop(0, (_CHUNKS_PER_TILE - 1) // 2)
        def _(i):
            j0 = 2 * i
            unpack(j0 + 1, 1)
            g_copy(1).start()
            g_copy(0).wait()
            s_sync(0)

            @pl.when(j0 + 2 < _CHUNKS_PER_TILE)
            def _():
                unpack(j0 + 2, 0)
                g_copy(0).start()

            g_copy(1).wait()
            s_sync(1)

        g_copy(0).wait()
        s_sync(0)

        plsc.subcore_barrier()

        pltpu.sync_copy(acc.at[pl.ds(base, _STRIPE)],
                        out_hbm.at[c].at[pl.ds(base, _STRIPE)])

    return ksc(h, epk)


def _gin_layer_tc(h, agg, w1, b1, w2, b2, gamma, beta, relu_out):
    def body(h_ref, a_ref, w1_ref, b1_ref, w2_ref, b2_ref, g_ref, be_ref, o_ref):
        t = h_ref[...] + a_ref[0, :_N] + a_ref[1, :_N]
        u = jnp.dot(t, w1_ref[...], preferred_element_type=jnp.float32) + b1_ref[...]
        u = jnp.maximum(u, 0.0)
        v = jnp.dot(u, w2_ref[...], preferred_element_type=jnp.float32) + b2_ref[...]
        mean = jnp.mean(v, axis=0, keepdims=True)
        cen = v - mean
        var = jnp.mean(cen * cen, axis=0, keepdims=True)
        o = cen * (g_ref[...] * lax.rsqrt(var + _EPS)) + be_ref[...]
        if relu_out:
            o = jnp.maximum(o, 0.0)
        o_ref[...] = o

    return pl.pallas_call(
        body, out_shape=jax.ShapeDtypeStruct((_N, _D), jnp.float32)
    )(h, agg, w1, b1, w2, b2, gamma, beta)


def _pool_linear_tc(h, batch2d, lin_w, lin_b):
    def body(h_ref, b_ref, w_ref, bias_ref, o_ref):
        ids = lax.broadcasted_iota(jnp.int32, (_N, _G), 1)
        oh = (b_ref[...] == ids).astype(jnp.float32)
        sums = lax.dot_general(oh, h_ref[...], (((0,), (0,)), ((), ())),
                               preferred_element_type=jnp.float32,
                               precision=lax.Precision.HIGHEST)
        cnt = jnp.sum(oh, axis=0)[:, None]
        pooled = sums / jnp.maximum(cnt, 1.0)
        o_ref[...] = jnp.dot(pooled, w_ref[...],
                             preferred_element_type=jnp.float32) + bias_ref[...]

    return pl.pallas_call(
        body, out_shape=jax.ShapeDtypeStruct((_G, _D), jnp.float32)
    )(h, batch2d, lin_w, lin_b)


def kernel(x, edge_index, batch,
           w1_0, b1_0, w2_0, b2_0, gamma_0, beta_0,
           w1_1, b1_1, w2_1, b2_1, gamma_1, beta_1,
           w1_2, b1_2, w2_2, b2_2, gamma_2, beta_2,
           lin_w, lin_b):
    epk = (edge_index[0] | (edge_index[1] << 14)).reshape(
        _NW, _CHUNKS_PER_TILE, _CHUNK)
    batch2d = batch.reshape(_N, 1)

    layers = [
        (w1_0, b1_0, w2_0, b2_0, gamma_0, beta_0),
        (w1_1, b1_1, w2_1, b2_1, gamma_1, beta_1),
        (w1_2, b1_2, w2_2, b2_2, gamma_2, beta_2),
    ]
    h = x
    for i, (w1, b1, w2, b2, g, be) in enumerate(layers):
        agg = _segment_sum_sc(h, epk)
        h = _gin_layer_tc(h, agg, w1, b1.reshape(1, _D), w2, b2.reshape(1, _D),
                          g.reshape(1, _D), be.reshape(1, _D), relu_out=(i < 2))
    return _pool_linear_tc(h, batch2d, lin_w, lin_b)
